# baseline (speedup 1.0000x reference)
import jax
import jax.numpy as jnp
from jax.experimental import pallas as pl
from jax.experimental.pallas import tpu as pltpu

BM = 200
NBUF = 5


def _gcn_manual_body(adj_hbm, x_ref, w_ref, b_ref, out_ref, buf, sem):
    n = x_ref.shape[0]
    nsteps = n // BM

    def copy(i, slot):
        return pltpu.make_async_copy(
            adj_hbm.at[pl.ds(i * BM, BM), :],
            buf.at[slot],
            sem.at[slot],
        )

    for s in range(NBUF):
        copy(s, s).start()

    wt = w_ref[...].T
    b = b_ref[...]

    def step(i, carry):
        slot = jax.lax.rem(i, NBUF)
        copy(i, slot).wait()
        agg = jnp.dot(buf[slot], x_ref[...], preferred_element_type=jnp.float32)
        out_ref[pl.ds(i * BM, BM), :] = (
            jnp.dot(agg, wt, preferred_element_type=jnp.float32) + b
        )

        @pl.when(i + NBUF < nsteps)
        def _():
            copy(i + NBUF, slot).start()

        return carry

    jax.lax.fori_loop(0, nsteps, step, 0)


def kernel(x, adj, W, bias):
    n, d_in = x.shape
    d_out = W.shape[0]

    out = pl.pallas_call(
        _gcn_manual_body,
        in_specs=[
            pl.BlockSpec(memory_space=pltpu.HBM),
            pl.BlockSpec((n, d_in), lambda: (0, 0)),
            pl.BlockSpec((d_out, d_in), lambda: (0, 0)),
            pl.BlockSpec((1, d_out), lambda: (0, 0)),
        ],
        out_specs=pl.BlockSpec((n, d_out), lambda: (0, 0)),
        out_shape=jax.ShapeDtypeStruct((n, d_out), jnp.float32),
        scratch_shapes=[
            pltpu.VMEM((NBUF, BM, n), jnp.float32),
            pltpu.SemaphoreType.DMA((NBUF,)),
        ],
        compiler_params=pltpu.CompilerParams(
            vmem_limit_bytes=62 * 1024 * 1024,
        ),
    )(adj, x, W, bias.reshape(1, d_out))
    return out
